# baseline (speedup 1.0000x reference)
import functools

import numpy as np
import jax
import jax.numpy as jnp
from jax import lax
from jax.experimental import pallas as pl
from jax.experimental.pallas import tpu as pltpu
from jax.experimental.pallas import tpu_sc as plsc

B, L, V = 1024, 50, 64
P = 0.05
N = B * L
R = L * V
NC, NS = 2, 16
NW = NC * NS
LB = 128
MAXL = 13
SCALE_S = 1.0 / (V - 2)
SCALE_G = float(V - 1) / (V - 2)
PR = float(P / (V - 2))


def _mask_t() -> np.ndarray:
    mask = np.random.RandomState(42).rand(N, V - 1) < P
    a = np.zeros((B, L, V), np.uint32)
    a[:, :, : V - 1] = mask.reshape(B, L, V - 1)
    at = np.ascontiguousarray(a.transpose(1, 2, 0))
    w = np.zeros((L, 2, B), np.uint32)
    for vv in range(32):
        w[:, 0, :] |= at[:, vv, :] << vv
        w[:, 1, :] |= at[:, vv + 32, :] << vv
    return w.reshape(L * 2, B).view(np.int32)


_AT = _mask_t()


def _sc_messages_t(m_t, a_t):
    mesh = plsc.VectorSubcoreMesh(core_axis_name="c", subcore_axis_name="s")

    def compute(l, mbuf, abuf, obuf):
        def group_body(g, carry):
            sl = pl.ds(g * 16, 16)
            one = jnp.ones((16,), jnp.int32)
            wlo = abuf[2 * l, sl]
            whi = abuf[2 * l + 1, sl]

            def bit(w, vv):
                return ((lax.shift_right_logical(w, jnp.full((16,), vv, jnp.int32)) & one)
                        .astype(jnp.float32))

            ghi = mbuf[31, sl] * bit(wlo, 31)
            glo = None
            s = jnp.zeros((16,), jnp.float32)
            for vv in range(32):
                mlo = mbuf[vv, sl]
                mhi = mbuf[vv + 32, sl]
                if vv == 0:
                    obuf[vv, sl] = mlo
                else:
                    obuf[vv, sl] = mlo - SCALE_G * glo
                obuf[vv + 32, sl] = mhi - SCALE_G * ghi
                glo = mlo * bit(wlo, vv)
                ghi = mhi * bit(whi, vv)
                s = s + glo + ghi
            sv = s * SCALE_S
            for v in range(1, V):
                plsc.addupdate(obuf.at[v, sl], sv)
            return carry

        lax.fori_loop(0, LB // 16, group_body, 0)

    @functools.partial(
        pl.kernel,
        out_type=jax.ShapeDtypeStruct((R, B), jnp.float32),
        mesh=mesh,
        scratch_types=[
            pltpu.VMEM((V, LB), jnp.float32),
            pltpu.VMEM((V, LB), jnp.float32),
            pltpu.VMEM((2 * L, LB), jnp.int32),
            pltpu.VMEM((V, LB), jnp.float32),
            pltpu.VMEM((V, LB), jnp.float32),
            pltpu.SemaphoreType.DMA,
            pltpu.SemaphoreType.DMA,
            pltpu.SemaphoreType.DMA,
            pltpu.SemaphoreType.DMA,
        ],
        compiler_params=pltpu.CompilerParams(needs_layout_passes=False),
    )
    def k(m_hbm, a_hbm, out_hbm, mb0, mb1, abuf, ob0, ob1, semA, semB, semO0, semO1):
        wid = lax.axis_index("s") * NC + lax.axis_index("c")
        lg = wid // 8
        b0 = (wid % 8) * LB
        l_start = MAXL * lg - jnp.maximum(lg - 2, 0)
        l_len = MAXL - (lg >= 2).astype(jnp.int32)

        def issue(st, mb, sem):
            l = l_start + st
            pltpu.async_copy(
                m_hbm.at[pl.ds(l * V, V // 2), pl.ds(b0, LB)],
                mb.at[pl.ds(0, V // 2)], sem)
            pltpu.async_copy(
                m_hbm.at[pl.ds(l * V + V // 2, V // 2), pl.ds(b0, LB)],
                mb.at[pl.ds(V // 2, V // 2)], sem)

        def drain(st, mb, sem):
            l = l_start + st
            pltpu.make_async_copy(
                m_hbm.at[pl.ds(l * V, V // 2), pl.ds(b0, LB)],
                mb.at[pl.ds(0, V // 2)], sem).wait()
            pltpu.make_async_copy(
                m_hbm.at[pl.ds(l * V + V // 2, V // 2), pl.ds(b0, LB)],
                mb.at[pl.ds(V // 2, V // 2)], sem).wait()

        def issue_out(st, ob, sem):
            l = l_start + st
            pltpu.async_copy(ob, out_hbm.at[pl.ds(l * V, V), pl.ds(b0, LB)], sem)

        def drain_out(st, ob, sem):
            l = l_start + st
            pltpu.make_async_copy(
                ob, out_hbm.at[pl.ds(l * V, V), pl.ds(b0, LB)], sem).wait()

        ca = pltpu.async_copy(a_hbm.at[:, pl.ds(b0, LB)], abuf, semA)
        issue(0, mb0, semA)
        ca.wait()

        def t_body(t, carry):
            st0 = 2 * t
            st1 = st0 + 1

            @pl.when(st1 < l_len)
            def _():
                issue(st1, mb1, semB)

            @pl.when(st0 < l_len)
            def _():
                drain(st0, mb0, semA)

                @pl.when(st0 >= 2)
                def _():
                    drain_out(st0 - 2, ob0, semO0)

                compute(l_start + st0, mb0, abuf, ob0)

            @pl.when(st0 + 2 < l_len)
            def _():
                issue(st0 + 2, mb0, semA)

            @pl.when(st0 < l_len)
            def _():
                issue_out(st0, ob0, semO0)

            @pl.when(st1 < l_len)
            def _():
                drain(st1, mb1, semB)

                @pl.when(st1 >= 2)
                def _():
                    drain_out(st1 - 2, ob1, semO1)

                compute(l_start + st1, mb1, abuf, ob1)
                issue_out(st1, ob1, semO1)

            return carry

        lax.fori_loop(0, (MAXL + 1) // 2, t_body, 0)

        @pl.when(l_len == MAXL)
        def _():
            drain_out(MAXL - 1, ob0, semO0)
            drain_out(MAXL - 2, ob1, semO1)

        @pl.when(l_len == MAXL - 1)
        def _():
            drain_out(MAXL - 3, ob0, semO0)
            drain_out(MAXL - 2, ob1, semO1)

    return k(m_t, a_t)


def _tc_logits_t(m_t, l_t):
    BB = 256

    def body(m_ref, l_ref, o_ref, om_ref, ol_ref):
        l = l_ref[...]
        e = jnp.exp(l)
        e0 = e[:, 0:1, :]
        q = (1.0 - P) * e + PR * jnp.clip(1.0 - e - e0, 0.0, 1.0)
        col = lax.broadcasted_iota(jnp.int32, l.shape, 1)
        o_ref[...] = jnp.where(col == 0, l, jnp.log(q))
        om_ref[...] = m_ref[...]
        ol_ref[...] = l

    spec = pl.BlockSpec((L, V, BB), lambda i: (0, 0, i))
    return pl.pallas_call(
        body,
        grid=(B // BB,),
        in_specs=[spec, spec],
        out_specs=[spec, spec, spec],
        out_shape=[jax.ShapeDtypeStruct((L, V, B), jnp.float32)] * 3,
    )(m_t, l_t)


def kernel(messages, logits):
    m_t = jnp.transpose(messages, (1, 2, 0))
    l_t = jnp.transpose(logits, (1, 2, 0))
    ln_t, mcp_t, lcp_t = _tc_logits_t(m_t, l_t)
    mn2d = _sc_messages_t(m_t.reshape(R, B), jnp.asarray(_AT))
    mn = jnp.transpose(mn2d.reshape(L, V, B), (2, 0, 1))
    ln = jnp.transpose(ln_t, (2, 0, 1))
    return (mn, ln, jnp.transpose(mcp_t, (2, 0, 1)), jnp.transpose(lcp_t, (2, 0, 1)))


# speedup vs baseline: 1.0069x; 1.0069x over previous
import functools

import numpy as np
import jax
import jax.numpy as jnp
from jax import lax
from jax.experimental import pallas as pl
from jax.experimental.pallas import tpu as pltpu
from jax.experimental.pallas import tpu_sc as plsc

B, L, V = 1024, 50, 64
P = 0.05
N = B * L
R = L * V
NC, NS = 2, 16
NW = NC * NS
LB = 128
MAXL = 13
SCALE_S = 1.0 / (V - 2)
SCALE_G = float(V - 1) / (V - 2)
PR = float(P / (V - 2))


def _mask_t() -> np.ndarray:
    mask = np.random.RandomState(42).rand(N, V - 1) < P
    a = np.zeros((B, L, V), np.uint32)
    a[:, :, : V - 1] = mask.reshape(B, L, V - 1)
    at = np.ascontiguousarray(a.transpose(1, 2, 0))
    w = np.zeros((L, 2, B), np.uint32)
    for vv in range(32):
        w[:, 0, :] |= at[:, vv, :] << vv
        w[:, 1, :] |= at[:, vv + 32, :] << vv
    return w.reshape(L * 2, B).view(np.int32)


_AT = _mask_t()


def _sc_messages_t(m_t, a_t):
    mesh = plsc.VectorSubcoreMesh(core_axis_name="c", subcore_axis_name="s")

    def compute(l, mbuf, abuf, obuf):
        def group_body(g, carry):
            sl = pl.ds(g * 16, 16)
            one = jnp.ones((16,), jnp.int32)
            wlo = abuf[2 * l, sl]
            whi = abuf[2 * l + 1, sl]

            def bit(w, vv):
                return ((lax.shift_right_logical(w, jnp.full((16,), vv, jnp.int32)) & one)
                        .astype(jnp.float32))

            ghi = mbuf[31, sl] * bit(wlo, 31)
            glo = None
            s = jnp.zeros((16,), jnp.float32)
            for vv in range(32):
                mlo = mbuf[vv, sl]
                mhi = mbuf[vv + 32, sl]
                if vv == 0:
                    obuf[vv, sl] = mlo
                else:
                    obuf[vv, sl] = mlo - SCALE_G * glo
                obuf[vv + 32, sl] = mhi - SCALE_G * ghi
                glo = mlo * bit(wlo, vv)
                ghi = mhi * bit(whi, vv)
                s = s + glo + ghi
            sv = s * SCALE_S
            for v in range(1, V):
                plsc.addupdate(obuf.at[v, sl], sv)
            return carry

        lax.fori_loop(0, LB // 16, group_body, 0)

    @functools.partial(
        pl.kernel,
        out_type=jax.ShapeDtypeStruct((R, B), jnp.float32),
        mesh=mesh,
        scratch_types=[
            pltpu.VMEM((V, LB), jnp.float32),
            pltpu.VMEM((V, LB), jnp.float32),
            pltpu.VMEM((2 * L, LB), jnp.int32),
            pltpu.VMEM((V, LB), jnp.float32),
            pltpu.VMEM((V, LB), jnp.float32),
            pltpu.SemaphoreType.DMA,
            pltpu.SemaphoreType.DMA,
            pltpu.SemaphoreType.DMA,
            pltpu.SemaphoreType.DMA,
        ],
        compiler_params=pltpu.CompilerParams(needs_layout_passes=False),
    )
    def k(m_hbm, a_hbm, out_hbm, mb0, mb1, abuf, ob0, ob1, semA, semB, semO0, semO1):
        wid = lax.axis_index("s") * NC + lax.axis_index("c")
        lg = wid // 8
        b0 = (wid % 8) * LB
        l_start = MAXL * lg - jnp.maximum(lg - 2, 0)
        l_len = MAXL - (lg >= 2).astype(jnp.int32)

        def issue(st, mb, sem):
            l = l_start + st
            pltpu.async_copy(m_hbm.at[pl.ds(l * V, V), pl.ds(b0, LB)], mb, sem)

        def drain(st, mb, sem):
            l = l_start + st
            pltpu.make_async_copy(
                m_hbm.at[pl.ds(l * V, V), pl.ds(b0, LB)], mb, sem).wait()

        def issue_out(st, ob, sem):
            l = l_start + st
            pltpu.async_copy(ob, out_hbm.at[pl.ds(l * V, V), pl.ds(b0, LB)], sem)

        def drain_out(st, ob, sem):
            l = l_start + st
            pltpu.make_async_copy(
                ob, out_hbm.at[pl.ds(l * V, V), pl.ds(b0, LB)], sem).wait()

        ca = pltpu.async_copy(a_hbm.at[:, pl.ds(b0, LB)], abuf, semA)
        issue(0, mb0, semA)
        ca.wait()

        def t_body(t, carry):
            st0 = 2 * t
            st1 = st0 + 1

            @pl.when(st1 < l_len)
            def _():
                issue(st1, mb1, semB)

            @pl.when(st0 < l_len)
            def _():
                drain(st0, mb0, semA)

                @pl.when(st0 >= 2)
                def _():
                    drain_out(st0 - 2, ob0, semO0)

                compute(l_start + st0, mb0, abuf, ob0)

            @pl.when(st0 + 2 < l_len)
            def _():
                issue(st0 + 2, mb0, semA)

            @pl.when(st0 < l_len)
            def _():
                issue_out(st0, ob0, semO0)

            @pl.when(st1 < l_len)
            def _():
                drain(st1, mb1, semB)

                @pl.when(st1 >= 2)
                def _():
                    drain_out(st1 - 2, ob1, semO1)

                compute(l_start + st1, mb1, abuf, ob1)
                issue_out(st1, ob1, semO1)

            return carry

        lax.fori_loop(0, (MAXL + 1) // 2, t_body, 0)

        @pl.when(l_len == MAXL)
        def _():
            drain_out(MAXL - 1, ob0, semO0)
            drain_out(MAXL - 2, ob1, semO1)

        @pl.when(l_len == MAXL - 1)
        def _():
            drain_out(MAXL - 3, ob0, semO0)
            drain_out(MAXL - 2, ob1, semO1)

    return k(m_t, a_t)


def _tc_logits_t(m_t, l_t):
    BB = 256

    def body(m_ref, l_ref, o_ref, om_ref, ol_ref):
        l = l_ref[...]
        e = jnp.exp(l)
        e0 = e[:, 0:1, :]
        q = (1.0 - P) * e + PR * jnp.clip(1.0 - e - e0, 0.0, 1.0)
        col = lax.broadcasted_iota(jnp.int32, l.shape, 1)
        o_ref[...] = jnp.where(col == 0, l, jnp.log(q))
        om_ref[...] = m_ref[...]
        ol_ref[...] = l

    spec = pl.BlockSpec((L, V, BB), lambda i: (0, 0, i))
    return pl.pallas_call(
        body,
        grid=(B // BB,),
        in_specs=[spec, spec],
        out_specs=[spec, spec, spec],
        out_shape=[jax.ShapeDtypeStruct((L, V, B), jnp.float32)] * 3,
    )(m_t, l_t)


def kernel(messages, logits):
    m_t = jnp.transpose(messages, (1, 2, 0))
    l_t = jnp.transpose(logits, (1, 2, 0))
    ln_t, mcp_t, lcp_t = _tc_logits_t(m_t, l_t)
    mn2d = _sc_messages_t(m_t.reshape(R, B), jnp.asarray(_AT))
    mn = jnp.transpose(mn2d.reshape(L, V, B), (2, 0, 1))
    ln = jnp.transpose(ln_t, (2, 0, 1))
    return (mn, ln, jnp.transpose(mcp_t, (2, 0, 1)), jnp.transpose(lcp_t, (2, 0, 1)))
